# Initial kernel scaffold; baseline (speedup 1.0000x reference)
import jax
import jax.numpy as jnp
from jax.experimental import pallas as pl


def kernel(edge_index, W1, al1, ar1, b1, W2, al2, ar2, b2, Wc, bc):
    raise NotImplementedError("write your pallas kernel here")


# speedup vs baseline: 286.4228x; 286.4228x over previous
import functools

import jax
import jax.numpy as jnp
from jax import lax
from jax.experimental import pallas as pl
from jax.experimental.pallas import tpu as pltpu
from jax.experimental.pallas import tpu_sc as plsc

N = 50000
E = 800000
H = 2
D = 32

NCORE = 2
NSUB = 16
NW = NCORE * NSUB

R = 40
KCH = 5
R2 = 8
KCH2 = 25
PER_W = R * 128 * KCH
EPAD = PER_W * NW
ER = EPAD // 128
RPW = PER_W // 128
NPAD = 51200
ZSL = NPAD // NSUB
PADN = N

f32 = jnp.float32
i32 = jnp.int32

_MESH = plsc.VectorSubcoreMesh(core_axis_name="c", subcore_axis_name="s")
_CP = pltpu.CompilerParams(needs_layout_passes=False)


def _vc(v):
    return jnp.full((16,), v, f32)


def _wid():
    return lax.axis_index("c") * NSUB + lax.axis_index("s")


def _zero_shared(zbuf, accs):
    def zb(i, c):
        zbuf[pl.ds(i * 16, 16)] = jnp.zeros((16,), f32)
        return c

    lax.fori_loop(0, ZSL // 16, zb, 0)
    s = lax.axis_index("s")
    for acc in accs:
        pltpu.sync_copy(zbuf, acc.at[pl.ds(s * ZSL, ZSL)])


def _scatter_add_rows(vals, idx, acc, sem, nrows):
    descs = [
        pltpu.async_copy(vals.at[j], acc.at[idx.at[j]], sem, add=True)
        for j in range(nrows)
    ]
    for d in descs:
        d.wait()


@functools.partial(
    pl.kernel,
    out_type=jax.ShapeDtypeStruct((NCORE * NPAD,), f32),
    mesh=_MESH,
    compiler_params=_CP,
    scratch_types=[
        pltpu.VMEM((R, 128), i32),
        pltpu.VMEM((R, 128), f32),
        pltpu.VMEM((ZSL,), f32),
        pltpu.VMEM_SHARED((NPAD,), f32),
        pltpu.SemaphoreType.DMA,
    ],
)
def _deg_pass(dst_hbm, ones_hbm, out_hbm, idv, onev, zbuf, acc, sem):
    cid = lax.axis_index("c")
    sid = lax.axis_index("s")
    _zero_shared(zbuf, [acc])
    pltpu.sync_copy(ones_hbm, onev)
    plsc.subcore_barrier()
    rowbase = _wid() * RPW

    def chunk(k, c):
        pltpu.sync_copy(dst_hbm.at[pl.ds(rowbase + k * R, R)], idv)
        _scatter_add_rows(onev, idv, acc, sem, R)
        return c

    lax.fori_loop(0, KCH, chunk, 0)
    plsc.subcore_barrier()

    @pl.when(sid == 0)
    def _():
        pltpu.sync_copy(acc, out_hbm.at[pl.ds(cid * NPAD, NPAD)])


@functools.partial(
    pl.kernel,
    out_type=(
        jax.ShapeDtypeStruct((NCORE * NPAD,), f32),
        jax.ShapeDtypeStruct((NCORE * NPAD,), f32),
        jax.ShapeDtypeStruct((ER, 128), f32),
        jax.ShapeDtypeStruct((ER, 128), f32),
    ),
    mesh=_MESH,
    compiler_params=_CP,
    scratch_types=[
        pltpu.VMEM((NPAD,), f32),
        pltpu.VMEM((6 * 16,), f32),
        pltpu.VMEM((R, 128), i32),
        pltpu.VMEM((R, 128), i32),
        pltpu.VMEM((R, 128), f32),
        pltpu.VMEM((R, 128), f32),
        pltpu.VMEM((R, 128), f32),
        pltpu.VMEM((R, 128), f32),
        pltpu.VMEM((ZSL,), f32),
        pltpu.VMEM_SHARED((NPAD,), f32),
        pltpu.VMEM_SHARED((NPAD,), f32),
        pltpu.SemaphoreType.DMA,
    ],
)
def _s1_pass(src_hbm, dst_hbm, deg_hbm, par_hbm,
             s1p0_hbm, s1p1_hbm, t10_hbm, t11_hbm,
             degv, parv, isv, idv, ex0, ex1, tv0, tv1, zbuf,
             acc0, acc1, sem):
    cid = lax.axis_index("c")
    sid = lax.axis_index("s")
    _zero_shared(zbuf, [acc0, acc1])
    pltpu.sync_copy(deg_hbm, degv)
    pltpu.sync_copy(par_hbm, parv)
    plsc.subcore_barrier()
    rowbase = _wid() * RPW
    cl0 = parv[pl.ds(0, 16)]
    cl1 = parv[pl.ds(16, 16)]
    cr0 = parv[pl.ds(32, 16)]
    cr1 = parv[pl.ds(48, 16)]
    m0 = parv[pl.ds(64, 16)]
    m1 = parv[pl.ds(80, 16)]

    def chunk(k, c):
        rb = rowbase + k * R
        pltpu.sync_copy(src_hbm.at[pl.ds(rb, R)], isv)
        pltpu.sync_copy(dst_hbm.at[pl.ds(rb, R)], idv)

        def row(j, cj):
            def lane(m, cm):
                sl = pl.ds(m * 16, 16)
                s16 = isv[j, sl]
                d16 = idv[j, sl]
                degs = plsc.load_gather(degv, [s16])
                degd = plsc.load_gather(degv, [d16])
                x0 = degs * cl0 + degd * cr0
                e0 = jnp.maximum(x0, x0 * _vc(0.2))
                v0 = jnp.exp(e0 - m0)
                x1 = degs * cl1 + degd * cr1
                e1 = jnp.maximum(x1, x1 * _vc(0.2))
                v1 = jnp.exp(e1 - m1)
                ex0[j, sl] = v0
                ex1[j, sl] = v1
                tv0[j, sl] = degs * v0
                tv1[j, sl] = degs * v1
                return cm

            lax.fori_loop(0, 8, lane, 0)
            return cj

        lax.fori_loop(0, R, row, 0)
        _scatter_add_rows(ex0, idv, acc0, sem, R)
        _scatter_add_rows(ex1, idv, acc1, sem, R)
        pltpu.sync_copy(tv0, t10_hbm.at[pl.ds(rb, R)])
        pltpu.sync_copy(tv1, t11_hbm.at[pl.ds(rb, R)])
        return c

    lax.fori_loop(0, KCH, chunk, 0)
    plsc.subcore_barrier()

    @pl.when(sid == 0)
    def _():
        pltpu.sync_copy(acc0, s1p0_hbm.at[pl.ds(cid * NPAD, NPAD)])
        pltpu.sync_copy(acc1, s1p1_hbm.at[pl.ds(cid * NPAD, NPAD)])


@functools.partial(
    pl.kernel,
    out_type=(
        jax.ShapeDtypeStruct((NCORE * NPAD,), f32),
        jax.ShapeDtypeStruct((NCORE * NPAD,), f32),
    ),
    mesh=_MESH,
    compiler_params=_CP,
    scratch_types=[
        pltpu.VMEM((NPAD,), f32),
        pltpu.VMEM((NPAD,), f32),
        pltpu.VMEM((R2, 128), i32),
        pltpu.VMEM((R2, 128), f32),
        pltpu.VMEM((R2, 128), f32),
        pltpu.VMEM((ZSL,), f32),
        pltpu.VMEM_SHARED((NPAD,), f32),
        pltpu.VMEM_SHARED((NPAD,), f32),
        pltpu.SemaphoreType.DMA,
    ],
)
def _w1_pass(dst_hbm, t10_hbm, t11_hbm, s10_hbm, s11_hbm,
             w1p0_hbm, w1p1_hbm,
             s1v0, s1v1, idv, tv, wv, zbuf, acc0, acc1, sem):
    cid = lax.axis_index("c")
    sid = lax.axis_index("s")
    _zero_shared(zbuf, [acc0, acc1])
    pltpu.sync_copy(s10_hbm, s1v0)
    pltpu.sync_copy(s11_hbm, s1v1)
    plsc.subcore_barrier()
    rowbase = _wid() * RPW

    def chunk(k, c):
        rb = rowbase + k * R2
        pltpu.sync_copy(dst_hbm.at[pl.ds(rb, R2)], idv)
        for s1v, t_hbm, acc in ((s1v0, t10_hbm, acc0), (s1v1, t11_hbm, acc1)):
            pltpu.sync_copy(t_hbm.at[pl.ds(rb, R2)], tv)

            def row(j, cj):
                def lane(m, cm):
                    sl = pl.ds(m * 16, 16)
                    d16 = idv[j, sl]
                    sd = plsc.load_gather(s1v, [d16])
                    wv[j, sl] = tv[j, sl] / jnp.maximum(sd, _vc(1e-9))
                    return cm

                lax.fori_loop(0, 8, lane, 0)
                return cj

            lax.fori_loop(0, R2, row, 0)
            _scatter_add_rows(wv, idv, acc, sem, R2)
        return c

    lax.fori_loop(0, KCH2, chunk, 0)
    plsc.subcore_barrier()

    @pl.when(sid == 0)
    def _():
        pltpu.sync_copy(acc0, w1p0_hbm.at[pl.ds(cid * NPAD, NPAD)])
        pltpu.sync_copy(acc1, w1p1_hbm.at[pl.ds(cid * NPAD, NPAD)])


@functools.partial(
    pl.kernel,
    out_type=(
        jax.ShapeDtypeStruct((NCORE * NPAD,), f32),
        jax.ShapeDtypeStruct((ER, 128), f32),
    ),
    mesh=_MESH,
    compiler_params=_CP,
    scratch_types=[
        pltpu.VMEM((NPAD,), f32),
        pltpu.VMEM((NPAD,), f32),
        pltpu.VMEM((16,), f32),
        pltpu.VMEM((R, 128), i32),
        pltpu.VMEM((R, 128), i32),
        pltpu.VMEM((R, 128), f32),
        pltpu.VMEM((ZSL,), f32),
        pltpu.VMEM_SHARED((NPAD,), f32),
        pltpu.SemaphoreType.DMA,
    ],
)
def _s2_pass(src_hbm, dst_hbm, el_hbm, er_hbm, par_hbm,
             s2p_hbm, ex2_hbm,
             elv, erv, parv, isv, idv, exv, zbuf, acc, sem):
    cid = lax.axis_index("c")
    sid = lax.axis_index("s")
    _zero_shared(zbuf, [acc])
    pltpu.sync_copy(el_hbm, elv)
    pltpu.sync_copy(er_hbm, erv)
    pltpu.sync_copy(par_hbm, parv)
    plsc.subcore_barrier()
    rowbase = _wid() * RPW
    m2 = parv[pl.ds(0, 16)]

    def chunk(k, c):
        rb = rowbase + k * R
        pltpu.sync_copy(src_hbm.at[pl.ds(rb, R)], isv)
        pltpu.sync_copy(dst_hbm.at[pl.ds(rb, R)], idv)

        def row(j, cj):
            def lane(m, cm):
                sl = pl.ds(m * 16, 16)
                s16 = isv[j, sl]
                d16 = idv[j, sl]
                x = plsc.load_gather(elv, [s16]) + plsc.load_gather(erv, [d16])
                e = jnp.maximum(x, x * _vc(0.2))
                exv[j, sl] = jnp.exp(e - m2)
                return cm

            lax.fori_loop(0, 8, lane, 0)
            return cj

        lax.fori_loop(0, R, row, 0)
        _scatter_add_rows(exv, idv, acc, sem, R)
        pltpu.sync_copy(exv, ex2_hbm.at[pl.ds(rb, R)])
        return c

    lax.fori_loop(0, KCH, chunk, 0)
    plsc.subcore_barrier()

    @pl.when(sid == 0)
    def _():
        pltpu.sync_copy(acc, s2p_hbm.at[pl.ds(cid * NPAD, NPAD)])


@functools.partial(
    pl.kernel,
    out_type=(
        jax.ShapeDtypeStruct((NCORE * NPAD,), f32),
        jax.ShapeDtypeStruct((NCORE * NPAD,), f32),
    ),
    mesh=_MESH,
    compiler_params=_CP,
    scratch_types=[
        pltpu.VMEM((NPAD,), f32),
        pltpu.VMEM((NPAD,), f32),
        pltpu.VMEM((R2, 128), i32),
        pltpu.VMEM((R2, 128), i32),
        pltpu.VMEM((R2, 128), f32),
        pltpu.VMEM((R2, 128), f32),
        pltpu.VMEM((ZSL,), f32),
        pltpu.VMEM_SHARED((NPAD,), f32),
        pltpu.VMEM_SHARED((NPAD,), f32),
        pltpu.SemaphoreType.DMA,
    ],
)
def _wsrc_pass(src_hbm, dst_hbm, ex20_hbm, ex21_hbm, s20_hbm, s21_hbm,
               wsp0_hbm, wsp1_hbm,
               s2v0, s2v1, isv, idv, exv, wv, zbuf, acc0, acc1, sem):
    cid = lax.axis_index("c")
    sid = lax.axis_index("s")
    _zero_shared(zbuf, [acc0, acc1])
    pltpu.sync_copy(s20_hbm, s2v0)
    pltpu.sync_copy(s21_hbm, s2v1)
    plsc.subcore_barrier()
    rowbase = _wid() * RPW

    def chunk(k, c):
        rb = rowbase + k * R2
        pltpu.sync_copy(src_hbm.at[pl.ds(rb, R2)], isv)
        pltpu.sync_copy(dst_hbm.at[pl.ds(rb, R2)], idv)
        for s2v, ex_hbm, acc in ((s2v0, ex20_hbm, acc0), (s2v1, ex21_hbm, acc1)):
            pltpu.sync_copy(ex_hbm.at[pl.ds(rb, R2)], exv)

            def row(j, cj):
                def lane(m, cm):
                    sl = pl.ds(m * 16, 16)
                    d16 = idv[j, sl]
                    sd = plsc.load_gather(s2v, [d16])
                    wv[j, sl] = exv[j, sl] / jnp.maximum(sd, _vc(1e-9))
                    return cm

                lax.fori_loop(0, 8, lane, 0)
                return cj

            lax.fori_loop(0, R2, row, 0)
            _scatter_add_rows(wv, isv, acc, sem, R2)
        return c

    lax.fori_loop(0, KCH2, chunk, 0)
    plsc.subcore_barrier()

    @pl.when(sid == 0)
    def _():
        pltpu.sync_copy(acc0, wsp0_hbm.at[pl.ds(cid * NPAD, NPAD)])
        pltpu.sync_copy(acc1, wsp1_hbm.at[pl.ds(cid * NPAD, NPAD)])


def _splat(vals):
    v = jnp.stack([v.astype(f32) for v in vals])
    return jnp.broadcast_to(v[:, None], (v.shape[0], 16)).reshape(-1)


def kernel(edge_index, W1, al1, ar1, b1, W2, al2, ar2, b2, Wc, bc):
    src = edge_index[0].astype(i32)
    dst = edge_index[1].astype(i32)
    pad = jnp.full((EPAD - E,), PADN, i32)
    src2 = jnp.concatenate([src, pad]).reshape(ER, 128)
    dst2 = jnp.concatenate([dst, pad]).reshape(ER, 128)
    ones = jnp.ones((R, 128), f32)

    def _comb(flat):
        return flat.reshape(NCORE, NPAD).sum(0)

    deg_valid = _comb(_deg_pass(dst2, ones))[:N]
    deg_pad = jnp.concatenate([deg_valid, jnp.zeros((NPAD - N,), f32)])

    w1r = W1.reshape(H, D)
    cl1 = (w1r * al1).sum(-1)
    cr1 = (w1r * ar1).sum(-1)
    maxdeg = deg_valid.max()
    m1 = maxdeg * (jax.nn.relu(cl1) + jax.nn.relu(cr1))
    par1 = _splat([cl1[0], cl1[1], cr1[0], cr1[1], m1[0], m1[1]])

    s1p0, s1p1, t10, t11 = _s1_pass(src2, dst2, deg_pad, par1)
    s10 = _comb(s1p0)
    s11 = _comb(s1p1)

    w1p0, w1p1 = _w1_pass(dst2, t10, t11, s10, s11)
    w1s0 = _comb(w1p0)
    w1s1 = _comb(w1p1)
    w1 = jnp.stack([w1s0, w1s1], axis=1)

    A = w1r / H
    cvec = b1.mean(0)
    W2r = W2.reshape(D, H, D)
    vl2 = jnp.einsum("khd,hd->kh", W2r, al2)
    vr2 = jnp.einsum("khd,hd->kh", W2r, ar2)
    P = A @ vl2
    p0 = cvec @ vl2
    Q = A @ vr2
    q0 = cvec @ vr2
    el2 = w1 @ P + p0
    er2 = w1 @ Q + q0
    m2 = el2.max(0) + er2.max(0)
    m2 = jnp.maximum(m2, 0.2 * m2)

    s2p0, ex20 = _s2_pass(src2, dst2, el2[:, 0].copy(), er2[:, 0].copy(),
                          _splat([m2[0]]))
    s2p1, ex21 = _s2_pass(src2, dst2, el2[:, 1].copy(), er2[:, 1].copy(),
                          _splat([m2[1]]))
    s20 = _comb(s2p0)
    s21 = _comb(s2p1)

    wsp0, wsp1 = _wsrc_pass(src2, dst2, ex20, ex21, s20, s21)
    wsrc = jnp.stack([_comb(wsp0), _comb(wsp1)], axis=1)[:N]

    w1v = w1[:N]
    S = wsrc.T @ w1v
    t = wsrc.sum(0)
    u = S @ A + t[:, None] * cvec[None, :]
    contrib = jnp.einsum("hk,khd->d", u, W2r)
    hg = contrib / (N * H) + b2.mean(0)
    return hg[None, :] @ Wc + bc


# speedup vs baseline: 367.5771x; 1.2833x over previous
import functools

import jax
import jax.numpy as jnp
from jax import lax
from jax.experimental import pallas as pl
from jax.experimental.pallas import tpu as pltpu
from jax.experimental.pallas import tpu_sc as plsc

N = 50000
E = 800000
H = 2
D = 32

NCORE = 2
NSUB = 16
NW = NCORE * NSUB

RPW = 200
PER_W = RPW * 128
EPAD = PER_W * NW
ER = EPAD // 128
R1 = 40
K1 = RPW // R1
R2 = 8
K2 = RPW // R2
NPAD = 51200
ZSL = NPAD // NSUB
PADN = N

f32 = jnp.float32
i32 = jnp.int32

_MESH = plsc.VectorSubcoreMesh(core_axis_name="c", subcore_axis_name="s")
_CP = pltpu.CompilerParams(needs_layout_passes=False)
_NODE_OUT = jax.ShapeDtypeStruct((NCORE * NPAD,), f32)


def _vc(v):
    return jnp.full((16,), v, f32)


def _wid():
    return lax.axis_index("c") * NSUB + lax.axis_index("s")


def _zero_shared(zsrc, accs):
    def zb(i, c):
        zsrc[pl.ds(i * 16, 16)] = jnp.zeros((16,), f32)
        return c

    lax.fori_loop(0, ZSL // 16, zb, 0)
    s = lax.axis_index("s")
    for acc in accs:
        pltpu.sync_copy(zsrc.at[pl.ds(0, ZSL)], acc.at[pl.ds(s * ZSL, ZSL)])


def _scatter_add_rows(vals, idx, acc, sem, nrows):
    return [
        pltpu.async_copy(vals.at[j], acc.at[idx.at[j]], sem, add=True)
        for j in range(nrows)
    ]


def _drain(descs):
    for d in descs:
        d.wait()


@functools.partial(
    pl.kernel,
    out_type=_NODE_OUT,
    mesh=_MESH,
    compiler_params=_CP,
    scratch_types=[
        pltpu.VMEM((RPW, 128), i32),
        pltpu.VMEM((RPW, 128), f32),
        pltpu.VMEM_SHARED((NPAD,), f32),
        pltpu.SemaphoreType.DMA,
    ],
)
def _deg_pass(dst_hbm, ones_hbm, out_hbm, idv, onev, acc, sem):
    cid = lax.axis_index("c")
    sid = lax.axis_index("s")
    _zero_shared(onev.at[0], [acc])
    pltpu.sync_copy(ones_hbm, onev)
    pltpu.sync_copy(dst_hbm.at[pl.ds(_wid() * RPW, RPW)], idv)
    plsc.subcore_barrier()
    _drain(_scatter_add_rows(onev, idv, acc, sem, RPW))
    plsc.subcore_barrier()

    @pl.when(sid == 0)
    def _():
        pltpu.sync_copy(acc, out_hbm.at[pl.ds(cid * NPAD, NPAD)])


@functools.partial(
    pl.kernel,
    out_type=(_NODE_OUT, _NODE_OUT, _NODE_OUT, _NODE_OUT),
    mesh=_MESH,
    compiler_params=_CP,
    scratch_types=[
        pltpu.VMEM((NPAD,), f32),
        pltpu.VMEM((6 * 16,), f32),
        pltpu.VMEM((R1, 128), i32),
        pltpu.VMEM((R1, 128), i32),
        pltpu.VMEM((R1, 128), f32),
        pltpu.VMEM((R1, 128), f32),
        pltpu.VMEM((R1, 128), f32),
        pltpu.VMEM((R1, 128), f32),
        pltpu.VMEM_SHARED((NPAD,), f32),
        pltpu.VMEM_SHARED((NPAD,), f32),
        pltpu.VMEM_SHARED((NPAD,), f32),
        pltpu.VMEM_SHARED((NPAD,), f32),
        pltpu.SemaphoreType.DMA,
    ],
)
def _layer1_pass(src_hbm, dst_hbm, deg_hbm, par_hbm,
                 s1p0_hbm, s1p1_hbm, ts0_hbm, ts1_hbm,
                 degv, parv, isv, idv, ex0, ex1, tv0, tv1,
                 acc_s0, acc_s1, acc_t0, acc_t1, sem):
    cid = lax.axis_index("c")
    sid = lax.axis_index("s")
    _zero_shared(degv, [acc_s0, acc_s1, acc_t0, acc_t1])
    pltpu.sync_copy(deg_hbm, degv)
    pltpu.sync_copy(par_hbm, parv)
    plsc.subcore_barrier()
    rowbase = _wid() * RPW
    cl0 = parv[pl.ds(0, 16)]
    cl1 = parv[pl.ds(16, 16)]
    cr0 = parv[pl.ds(32, 16)]
    cr1 = parv[pl.ds(48, 16)]
    m0 = parv[pl.ds(64, 16)]
    m1 = parv[pl.ds(80, 16)]

    def chunk1(k, ck):
        rb = rowbase + k * R1
        pltpu.sync_copy(src_hbm.at[pl.ds(rb, R1)], isv)
        pltpu.sync_copy(dst_hbm.at[pl.ds(rb, R1)], idv)

        def row(j, cj):
            def lane(m, cm):
                sl = pl.ds(m * 16, 16)
                s16 = isv[j, sl]
                d16 = idv[j, sl]
                degs = plsc.load_gather(degv, [s16])
                degd = plsc.load_gather(degv, [d16])
                x0 = degs * cl0 + degd * cr0
                e0 = jnp.maximum(x0, x0 * _vc(0.2))
                v0 = jnp.exp(e0 - m0)
                x1 = degs * cl1 + degd * cr1
                e1 = jnp.maximum(x1, x1 * _vc(0.2))
                v1 = jnp.exp(e1 - m1)
                ex0[j, sl] = v0
                ex1[j, sl] = v1
                tv0[j, sl] = degs * v0
                tv1[j, sl] = degs * v1
                return cm

            lax.fori_loop(0, 8, lane, 0)
            return cj

        lax.fori_loop(0, R1, row, 0)
        ds_ = _scatter_add_rows(ex0, idv, acc_s0, sem, R1)
        ds_ += _scatter_add_rows(ex1, idv, acc_s1, sem, R1)
        ds_ += _scatter_add_rows(tv0, idv, acc_t0, sem, R1)
        ds_ += _scatter_add_rows(tv1, idv, acc_t1, sem, R1)
        _drain(ds_)
        return ck

    lax.fori_loop(0, K1, chunk1, 0)
    plsc.subcore_barrier()

    @pl.when(sid == 0)
    def _():
        sl = pl.ds(cid * NPAD, NPAD)
        pltpu.sync_copy(acc_s0, s1p0_hbm.at[sl])
        pltpu.sync_copy(acc_s1, s1p1_hbm.at[sl])
        pltpu.sync_copy(acc_t0, ts0_hbm.at[sl])
        pltpu.sync_copy(acc_t1, ts1_hbm.at[sl])


@functools.partial(
    pl.kernel,
    out_type=(_NODE_OUT, _NODE_OUT, _NODE_OUT, _NODE_OUT, _NODE_OUT, _NODE_OUT),
    mesh=_MESH,
    compiler_params=_CP,
    scratch_types=[
        pltpu.VMEM((NPAD,), f32),
        pltpu.VMEM((NPAD,), f32),
        pltpu.VMEM((14 * 16,), f32),
        pltpu.VMEM((R2, 128), i32),
        pltpu.VMEM((R2, 128), i32),
        pltpu.VMEM((R2, 128), f32),
        pltpu.VMEM((R2, 128), f32),
        pltpu.VMEM((R2, 128), f32),
        pltpu.VMEM((R2, 128), f32),
        pltpu.VMEM((R2, 128), f32),
        pltpu.VMEM((R2, 128), f32),
        pltpu.VMEM_SHARED((NPAD,), f32),
        pltpu.VMEM_SHARED((NPAD,), f32),
        pltpu.VMEM_SHARED((NPAD,), f32),
        pltpu.VMEM_SHARED((NPAD,), f32),
        pltpu.VMEM_SHARED((NPAD,), f32),
        pltpu.VMEM_SHARED((NPAD,), f32),
        pltpu.SemaphoreType.DMA,
    ],
)
def _layer2_pass(src_hbm, dst_hbm, w0_hbm, w1_hbm, par_hbm,
                 s2p0_hbm, s2p1_hbm, g00_hbm, g01_hbm, g10_hbm, g11_hbm,
                 w0v, w1v, parv, isv, idv, ex0, ex1, g00, g01, g10, g11,
                 acc_e0, acc_e1, acc_g00, acc_g01, acc_g10, acc_g11, sem):
    cid = lax.axis_index("c")
    sid = lax.axis_index("s")
    _zero_shared(w0v, [acc_e0, acc_e1, acc_g00, acc_g01, acc_g10, acc_g11])
    pltpu.sync_copy(w0_hbm, w0v)
    pltpu.sync_copy(w1_hbm, w1v)
    pltpu.sync_copy(par_hbm, parv)
    plsc.subcore_barrier()
    rowbase = _wid() * RPW
    P00 = parv[pl.ds(0, 16)]
    P10 = parv[pl.ds(16, 16)]
    P01 = parv[pl.ds(32, 16)]
    P11 = parv[pl.ds(48, 16)]
    pl0 = parv[pl.ds(64, 16)]
    pl1 = parv[pl.ds(80, 16)]
    Q00 = parv[pl.ds(96, 16)]
    Q10 = parv[pl.ds(112, 16)]
    Q01 = parv[pl.ds(128, 16)]
    Q11 = parv[pl.ds(144, 16)]
    ql0 = parv[pl.ds(160, 16)]
    ql1 = parv[pl.ds(176, 16)]
    m0 = parv[pl.ds(192, 16)]
    m1 = parv[pl.ds(208, 16)]

    def chunk2(k, ck):
        rb = rowbase + k * R2
        pltpu.sync_copy(src_hbm.at[pl.ds(rb, R2)], isv)
        pltpu.sync_copy(dst_hbm.at[pl.ds(rb, R2)], idv)

        def row(j, cj):
            def lane(m, cm):
                sl = pl.ds(m * 16, 16)
                s16 = isv[j, sl]
                d16 = idv[j, sl]
                w0s = plsc.load_gather(w0v, [s16])
                w1s = plsc.load_gather(w1v, [s16])
                w0d = plsc.load_gather(w0v, [d16])
                w1d = plsc.load_gather(w1v, [d16])
                x0 = (w0s * P00 + w1s * P10 + pl0) + (w0d * Q00 + w1d * Q10 + ql0)
                e0 = jnp.maximum(x0, x0 * _vc(0.2))
                v0 = jnp.exp(e0 - m0)
                x1 = (w0s * P01 + w1s * P11 + pl1) + (w0d * Q01 + w1d * Q11 + ql1)
                e1 = jnp.maximum(x1, x1 * _vc(0.2))
                v1 = jnp.exp(e1 - m1)
                ex0[j, sl] = v0
                ex1[j, sl] = v1
                g00[j, sl] = v0 * w0s
                g01[j, sl] = v0 * w1s
                g10[j, sl] = v1 * w0s
                g11[j, sl] = v1 * w1s
                return cm

            lax.fori_loop(0, 8, lane, 0)
            return cj

        lax.fori_loop(0, R2, row, 0)
        ds_ = _scatter_add_rows(ex0, idv, acc_e0, sem, R2)
        ds_ += _scatter_add_rows(ex1, idv, acc_e1, sem, R2)
        ds_ += _scatter_add_rows(g00, idv, acc_g00, sem, R2)
        ds_ += _scatter_add_rows(g01, idv, acc_g01, sem, R2)
        ds_ += _scatter_add_rows(g10, idv, acc_g10, sem, R2)
        ds_ += _scatter_add_rows(g11, idv, acc_g11, sem, R2)
        _drain(ds_)
        return ck

    lax.fori_loop(0, K2, chunk2, 0)
    plsc.subcore_barrier()

    @pl.when(sid == 0)
    def _():
        sl = pl.ds(cid * NPAD, NPAD)
        pltpu.sync_copy(acc_e0, s2p0_hbm.at[sl])
        pltpu.sync_copy(acc_e1, s2p1_hbm.at[sl])
        pltpu.sync_copy(acc_g00, g00_hbm.at[sl])
        pltpu.sync_copy(acc_g01, g01_hbm.at[sl])
        pltpu.sync_copy(acc_g10, g10_hbm.at[sl])
        pltpu.sync_copy(acc_g11, g11_hbm.at[sl])


def _splat(vals):
    v = jnp.stack([v.astype(f32) for v in vals])
    return jnp.broadcast_to(v[:, None], (v.shape[0], 16)).reshape(-1)


def kernel(edge_index, W1, al1, ar1, b1, W2, al2, ar2, b2, Wc, bc):
    src = edge_index[0].astype(i32)
    dst = edge_index[1].astype(i32)
    pad = jnp.full((EPAD - E,), PADN, i32)
    src2 = jnp.concatenate([src, pad]).reshape(ER, 128)
    dst2 = jnp.concatenate([dst, pad]).reshape(ER, 128)
    ones = jnp.ones((RPW, 128), f32)

    def _comb(flat):
        return flat.reshape(NCORE, NPAD).sum(0)

    deg_valid = _comb(_deg_pass(dst2, ones))[:N]
    deg_pad = jnp.concatenate([deg_valid, jnp.zeros((NPAD - N,), f32)])

    w1r = W1.reshape(H, D)
    cl1 = (w1r * al1).sum(-1)
    cr1 = (w1r * ar1).sum(-1)
    maxdeg = deg_valid.max()
    m1 = maxdeg * (jax.nn.relu(cl1) + jax.nn.relu(cr1))
    par1 = _splat([cl1[0], cl1[1], cr1[0], cr1[1], m1[0], m1[1]])

    s1p0, s1p1, tsp0, tsp1 = _layer1_pass(src2, dst2, deg_pad, par1)
    w1s0 = _comb(tsp0) / jnp.maximum(_comb(s1p0), 1e-9)
    w1s1 = _comb(tsp1) / jnp.maximum(_comb(s1p1), 1e-9)
    w1 = jnp.stack([w1s0, w1s1], axis=1)

    A = w1r / H
    cvec = b1.mean(0)
    W2r = W2.reshape(D, H, D)
    vl2 = jnp.einsum("khd,hd->kh", W2r, al2)
    vr2 = jnp.einsum("khd,hd->kh", W2r, ar2)
    P = A @ vl2
    p0 = cvec @ vl2
    Q = A @ vr2
    q0 = cvec @ vr2
    el2 = w1 @ P + p0
    er2 = w1 @ Q + q0
    m2 = el2.max(0) + er2.max(0)
    m2 = jnp.maximum(m2, 0.2 * m2)
    par2 = _splat([P[0, 0], P[1, 0], P[0, 1], P[1, 1], p0[0], p0[1],
                   Q[0, 0], Q[1, 0], Q[0, 1], Q[1, 1], q0[0], q0[1],
                   m2[0], m2[1]])

    s2p0, s2p1, g00p, g01p, g10p, g11p = _layer2_pass(
        src2, dst2, w1s0, w1s1, par2)
    den0 = jnp.maximum(_comb(s2p0)[:N], 1e-9)
    den1 = jnp.maximum(_comb(s2p1)[:N], 1e-9)
    t = jnp.stack([(_comb(s2p0)[:N] / den0).sum(),
                   (_comb(s2p1)[:N] / den1).sum()])
    S = jnp.stack([
        jnp.stack([(_comb(g00p)[:N] / den0).sum(),
                   (_comb(g01p)[:N] / den0).sum()]),
        jnp.stack([(_comb(g10p)[:N] / den1).sum(),
                   (_comb(g11p)[:N] / den1).sum()]),
    ])

    u = S @ A + t[:, None] * cvec[None, :]
    contrib = jnp.einsum("hk,khd->d", u, W2r)
    hg = contrib / (N * H) + b2.mean(0)
    return hg[None, :] @ Wc + bc


# speedup vs baseline: 369.3369x; 1.0048x over previous
import functools

import jax
import jax.numpy as jnp
from jax import lax
from jax.experimental import pallas as pl
from jax.experimental.pallas import tpu as pltpu
from jax.experimental.pallas import tpu_sc as plsc

N = 50000
E = 800000
H = 2
D = 32

NCORE = 2
NSUB = 16
NW = NCORE * NSUB

RPW = 200
PER_W = RPW * 128
EPAD = PER_W * NW
ER = EPAD // 128
R1 = 40
K1 = RPW // R1
R2 = 8
K2 = RPW // R2
NPAD = 51200
ZSL = NPAD // NSUB
PADN = N

f32 = jnp.float32
i32 = jnp.int32

_MESH = plsc.VectorSubcoreMesh(core_axis_name="c", subcore_axis_name="s")
_CP = pltpu.CompilerParams(needs_layout_passes=False)
_NODE_OUT = jax.ShapeDtypeStruct((NCORE * NPAD,), f32)


def _vc(v):
    return jnp.full((16,), v, f32)


def _wid():
    return lax.axis_index("c") * NSUB + lax.axis_index("s")


def _zero_shared(zsrc, accs):
    def zb(i, c):
        zsrc[pl.ds(i * 16, 16)] = jnp.zeros((16,), f32)
        return c

    lax.fori_loop(0, ZSL // 16, zb, 0)
    s = lax.axis_index("s")
    for acc in accs:
        pltpu.sync_copy(zsrc.at[pl.ds(0, ZSL)], acc.at[pl.ds(s * ZSL, ZSL)])


def _scatter_add_rows(vals, idx, acc, sem, nrows):
    return [
        pltpu.async_copy(vals.at[j], acc.at[idx.at[j]], sem, add=True)
        for j in range(nrows)
    ]


def _drain(descs):
    for d in descs:
        d.wait()


@functools.partial(
    pl.kernel,
    out_type=_NODE_OUT,
    mesh=_MESH,
    compiler_params=_CP,
    scratch_types=[
        pltpu.VMEM((RPW, 128), i32),
        pltpu.VMEM((RPW, 128), f32),
        pltpu.VMEM_SHARED((NPAD,), f32),
        pltpu.SemaphoreType.DMA,
    ],
)
def _deg_pass(dst_hbm, ones_hbm, out_hbm, idv, onev, acc, sem):
    cid = lax.axis_index("c")
    sid = lax.axis_index("s")
    _zero_shared(onev.at[0], [acc])
    pltpu.sync_copy(ones_hbm, onev)
    pltpu.sync_copy(dst_hbm.at[pl.ds(_wid() * RPW, RPW)], idv)
    plsc.subcore_barrier()
    _drain(_scatter_add_rows(onev, idv, acc, sem, RPW))
    plsc.subcore_barrier()

    @pl.when(sid == 0)
    def _():
        pltpu.sync_copy(acc, out_hbm.at[pl.ds(cid * NPAD, NPAD)])


@functools.partial(
    pl.kernel,
    out_type=(_NODE_OUT, _NODE_OUT, _NODE_OUT, _NODE_OUT),
    mesh=_MESH,
    compiler_params=_CP,
    scratch_types=[
        pltpu.VMEM((NPAD,), f32),
        pltpu.VMEM((6 * 16,), f32),
        pltpu.VMEM((R1, 128), i32),
        pltpu.VMEM((R1, 128), i32),
        pltpu.VMEM((R1, 128), i32),
        pltpu.VMEM((R1, 128), f32),
        pltpu.VMEM((R1, 128), f32),
        pltpu.VMEM((R1, 128), f32),
        pltpu.VMEM((R1, 128), f32),
        pltpu.VMEM((R1, 128), f32),
        pltpu.VMEM((R1, 128), f32),
        pltpu.VMEM((R1, 128), f32),
        pltpu.VMEM((R1, 128), f32),
        pltpu.VMEM_SHARED((NPAD,), f32),
        pltpu.VMEM_SHARED((NPAD,), f32),
        pltpu.VMEM_SHARED((NPAD,), f32),
        pltpu.VMEM_SHARED((NPAD,), f32),
        pltpu.SemaphoreType.DMA,
    ],
)
def _layer1_pass(src_hbm, dst_hbm, deg_hbm, par_hbm,
                 s1p0_hbm, s1p1_hbm, ts0_hbm, ts1_hbm,
                 degv, parv, isv, idv_a, idv_b,
                 ex0_a, ex1_a, tv0_a, tv1_a, ex0_b, ex1_b, tv0_b, tv1_b,
                 acc_s0, acc_s1, acc_t0, acc_t1, sem):
    cid = lax.axis_index("c")
    sid = lax.axis_index("s")
    _zero_shared(degv, [acc_s0, acc_s1, acc_t0, acc_t1])
    pltpu.sync_copy(deg_hbm, degv)
    pltpu.sync_copy(par_hbm, parv)
    plsc.subcore_barrier()
    rowbase = _wid() * RPW
    cl0 = parv[pl.ds(0, 16)]
    cl1 = parv[pl.ds(16, 16)]
    cr0 = parv[pl.ds(32, 16)]
    cr1 = parv[pl.ds(48, 16)]
    m0 = parv[pl.ds(64, 16)]
    m1 = parv[pl.ds(80, 16)]
    sets = [(idv_a, ex0_a, ex1_a, tv0_a, tv1_a),
            (idv_b, ex0_b, ex1_b, tv0_b, tv1_b)]

    def load_compute(k, st):
        idv, ex0, ex1, tv0, tv1 = st
        rb = rowbase + k * R1
        pltpu.sync_copy(src_hbm.at[pl.ds(rb, R1)], isv)
        pltpu.sync_copy(dst_hbm.at[pl.ds(rb, R1)], idv)

        def row(j, cj):
            def lane(m, cm):
                sl = pl.ds(m * 16, 16)
                s16 = isv[j, sl]
                d16 = idv[j, sl]
                degs = plsc.load_gather(degv, [s16])
                degd = plsc.load_gather(degv, [d16])
                x0 = degs * cl0 + degd * cr0
                e0 = jnp.maximum(x0, x0 * _vc(0.2))
                v0 = jnp.exp(e0 - m0)
                x1 = degs * cl1 + degd * cr1
                e1 = jnp.maximum(x1, x1 * _vc(0.2))
                v1 = jnp.exp(e1 - m1)
                ex0[j, sl] = v0
                ex1[j, sl] = v1
                tv0[j, sl] = degs * v0
                tv1[j, sl] = degs * v1
                return cm

            lax.fori_loop(0, 8, lane, 0)
            return cj

        lax.fori_loop(0, R1, row, 0)

    load_compute(0, sets[0])
    for k in range(K1):
        idv, ex0, ex1, tv0, tv1 = sets[k % 2]
        ds_ = _scatter_add_rows(ex0, idv, acc_s0, sem, R1)
        ds_ += _scatter_add_rows(ex1, idv, acc_s1, sem, R1)
        ds_ += _scatter_add_rows(tv0, idv, acc_t0, sem, R1)
        ds_ += _scatter_add_rows(tv1, idv, acc_t1, sem, R1)
        if k + 1 < K1:
            load_compute(k + 1, sets[(k + 1) % 2])
        _drain(ds_)
    plsc.subcore_barrier()

    @pl.when(sid == 0)
    def _():
        sl = pl.ds(cid * NPAD, NPAD)
        pltpu.sync_copy(acc_s0, s1p0_hbm.at[sl])
        pltpu.sync_copy(acc_s1, s1p1_hbm.at[sl])
        pltpu.sync_copy(acc_t0, ts0_hbm.at[sl])
        pltpu.sync_copy(acc_t1, ts1_hbm.at[sl])


@functools.partial(
    pl.kernel,
    out_type=(_NODE_OUT, _NODE_OUT, _NODE_OUT),
    mesh=_MESH,
    compiler_params=_CP,
    scratch_types=[
        pltpu.VMEM((NPAD,), f32),
        pltpu.VMEM((NPAD,), f32),
        pltpu.VMEM((6 * 16,), f32),
        pltpu.VMEM((R2, 128), i32),
        pltpu.VMEM((R2, 128), i32),
        pltpu.VMEM((R2, 128), i32),
        pltpu.VMEM((R2, 128), f32),
        pltpu.VMEM((R2, 128), f32),
        pltpu.VMEM((R2, 128), f32),
        pltpu.VMEM((R2, 128), f32),
        pltpu.VMEM((R2, 128), f32),
        pltpu.VMEM((R2, 128), f32),
        pltpu.VMEM_SHARED((NPAD,), f32),
        pltpu.VMEM_SHARED((NPAD,), f32),
        pltpu.VMEM_SHARED((NPAD,), f32),
        pltpu.SemaphoreType.DMA,
    ],
)
def _l2head_pass(src_hbm, dst_hbm, w0_hbm, w1_hbm, par_hbm,
                 s2p_hbm, g0p_hbm, g1p_hbm,
                 w0v, w1v, parv, isv, idv_a, idv_b,
                 ex_a, g0_a, g1_a, ex_b, g0_b, g1_b,
                 acc_e, acc_g0, acc_g1, sem):
    cid = lax.axis_index("c")
    sid = lax.axis_index("s")
    _zero_shared(w0v, [acc_e, acc_g0, acc_g1])
    pltpu.sync_copy(w0_hbm, w0v)
    pltpu.sync_copy(w1_hbm, w1v)
    pltpu.sync_copy(par_hbm, parv)
    plsc.subcore_barrier()
    rowbase = _wid() * RPW
    ca = parv[pl.ds(0, 16)]
    cb = parv[pl.ds(16, 16)]
    cc = parv[pl.ds(32, 16)]
    cd = parv[pl.ds(48, 16)]
    cst = parv[pl.ds(64, 16)]
    mh = parv[pl.ds(80, 16)]
    sets = [(idv_a, ex_a, g0_a, g1_a), (idv_b, ex_b, g0_b, g1_b)]

    def load_compute(k, st):
        idv, exb, g0b, g1b = st
        rb = rowbase + k * R2
        pltpu.sync_copy(src_hbm.at[pl.ds(rb, R2)], isv)
        pltpu.sync_copy(dst_hbm.at[pl.ds(rb, R2)], idv)

        def row(j, cj):
            def lane(m, cm):
                sl = pl.ds(m * 16, 16)
                s16 = isv[j, sl]
                d16 = idv[j, sl]
                w0s = plsc.load_gather(w0v, [s16])
                w1s = plsc.load_gather(w1v, [s16])
                w0d = plsc.load_gather(w0v, [d16])
                w1d = plsc.load_gather(w1v, [d16])
                x = w0s * ca + w1s * cb + w0d * cc + w1d * cd + cst
                e = jnp.maximum(x, x * _vc(0.2))
                v = jnp.exp(e - mh)
                exb[j, sl] = v
                g0b[j, sl] = v * w0s
                g1b[j, sl] = v * w1s
                return cm

            lax.fori_loop(0, 8, lane, 0)
            return cj

        lax.fori_loop(0, R2, row, 0)

    load_compute(0, sets[0])
    for k in range(K2):
        idv, exb, g0b, g1b = sets[k % 2]
        ds_ = _scatter_add_rows(exb, idv, acc_e, sem, R2)
        ds_ += _scatter_add_rows(g0b, idv, acc_g0, sem, R2)
        ds_ += _scatter_add_rows(g1b, idv, acc_g1, sem, R2)
        if k + 1 < K2:
            load_compute(k + 1, sets[(k + 1) % 2])
        _drain(ds_)
    plsc.subcore_barrier()

    @pl.when(sid == 0)
    def _():
        sl = pl.ds(cid * NPAD, NPAD)
        pltpu.sync_copy(acc_e, s2p_hbm.at[sl])
        pltpu.sync_copy(acc_g0, g0p_hbm.at[sl])
        pltpu.sync_copy(acc_g1, g1p_hbm.at[sl])


def _splat(vals):
    v = jnp.stack([v.astype(f32) for v in vals])
    return jnp.broadcast_to(v[:, None], (v.shape[0], 16)).reshape(-1)


def kernel(edge_index, W1, al1, ar1, b1, W2, al2, ar2, b2, Wc, bc):
    src = edge_index[0].astype(i32)
    dst = edge_index[1].astype(i32)
    pad = jnp.full((EPAD - E,), PADN, i32)
    src2 = jnp.concatenate([src, pad]).reshape(ER, 128)
    dst2 = jnp.concatenate([dst, pad]).reshape(ER, 128)
    ones = jnp.ones((RPW, 128), f32)

    def _comb(flat):
        return flat.reshape(NCORE, NPAD).sum(0)

    deg_valid = _comb(_deg_pass(dst2, ones))[:N]
    deg_pad = jnp.concatenate([deg_valid, jnp.zeros((NPAD - N,), f32)])

    w1r = W1.reshape(H, D)
    cl1 = (w1r * al1).sum(-1)
    cr1 = (w1r * ar1).sum(-1)
    maxdeg = deg_valid.max()
    m1 = maxdeg * (jax.nn.relu(cl1) + jax.nn.relu(cr1))
    par1 = _splat([cl1[0], cl1[1], cr1[0], cr1[1], m1[0], m1[1]])

    s1p0, s1p1, tsp0, tsp1 = _layer1_pass(src2, dst2, deg_pad, par1)
    w1s0 = _comb(tsp0) / jnp.maximum(_comb(s1p0), 1e-9)
    w1s1 = _comb(tsp1) / jnp.maximum(_comb(s1p1), 1e-9)
    w1 = jnp.stack([w1s0, w1s1], axis=1)

    A = w1r / H
    cvec = b1.mean(0)
    W2r = W2.reshape(D, H, D)
    vl2 = jnp.einsum("khd,hd->kh", W2r, al2)
    vr2 = jnp.einsum("khd,hd->kh", W2r, ar2)
    P = A @ vl2
    p0 = cvec @ vl2
    Q = A @ vr2
    q0 = cvec @ vr2
    el2 = w1 @ P + p0
    er2 = w1 @ Q + q0
    m2 = el2.max(0) + er2.max(0)
    m2 = jnp.maximum(m2, 0.2 * m2)

    def par2(h):
        return _splat([P[0, h], P[1, h], Q[0, h], Q[1, h],
                       p0[h] + q0[h], m2[h]])

    s2p0, g00p, g01p = _l2head_pass(src2, dst2, w1s0, w1s1, par2(0))
    s2p1, g10p, g11p = _l2head_pass(src2, dst2, w1s0, w1s1, par2(1))
    den0 = jnp.maximum(_comb(s2p0)[:N], 1e-9)
    den1 = jnp.maximum(_comb(s2p1)[:N], 1e-9)
    t = jnp.stack([(_comb(s2p0)[:N] / den0).sum(),
                   (_comb(s2p1)[:N] / den1).sum()])
    S = jnp.stack([
        jnp.stack([(_comb(g00p)[:N] / den0).sum(),
                   (_comb(g01p)[:N] / den0).sum()]),
        jnp.stack([(_comb(g10p)[:N] / den1).sum(),
                   (_comb(g11p)[:N] / den1).sum()]),
    ])

    u = S @ A + t[:, None] * cvec[None, :]
    contrib = jnp.einsum("hk,khd->d", u, W2r)
    hg = contrib / (N * H) + b2.mean(0)
    return hg[None, :] @ Wc + bc


# speedup vs baseline: 374.8685x; 1.0150x over previous
import functools

import jax
import jax.numpy as jnp
from jax import lax
from jax.experimental import pallas as pl
from jax.experimental.pallas import tpu as pltpu
from jax.experimental.pallas import tpu_sc as plsc

N = 50000
E = 800000
H = 2
D = 32

NCORE = 2
NSUB = 16
NW = NCORE * NSUB

RPW = 200
PER_W = RPW * 128
EPAD = PER_W * NW
ER = EPAD // 128
R1 = 40
K1 = RPW // R1
R2 = 8
K2 = RPW // R2
NPAD = 50176
ZSL = NPAD // NSUB
PADN = N

f32 = jnp.float32
i32 = jnp.int32

_MESH = plsc.VectorSubcoreMesh(core_axis_name="c", subcore_axis_name="s")
_CP = pltpu.CompilerParams(needs_layout_passes=False)
_NODE_OUT = jax.ShapeDtypeStruct((NCORE * NPAD,), f32)
_NODE_OUT1 = jax.ShapeDtypeStruct((NPAD,), f32)


def _vc(v):
    return jnp.full((16,), v, f32)


def _wid():
    return lax.axis_index("c") * NSUB + lax.axis_index("s")


def _zero_shared(zsrc, accs):
    def zb(i, c):
        zsrc[pl.ds(i * 16, 16)] = jnp.zeros((16,), f32)
        return c

    lax.fori_loop(0, ZSL // 16, zb, 0)
    s = lax.axis_index("s")
    for acc in accs:
        pltpu.sync_copy(zsrc.at[pl.ds(0, ZSL)], acc.at[pl.ds(s * ZSL, ZSL)])


def _scatter_add_rows(vals, idx, acc, sem, nrows):
    return [
        pltpu.async_copy(vals.at[j], acc.at[idx.at[j]], sem, add=True)
        for j in range(nrows)
    ]


def _drain(descs):
    for d in descs:
        d.wait()


@functools.partial(
    pl.kernel,
    out_type=_NODE_OUT,
    mesh=_MESH,
    compiler_params=_CP,
    scratch_types=[
        pltpu.VMEM((RPW, 128), i32),
        pltpu.VMEM((RPW, 128), f32),
        pltpu.VMEM_SHARED((NPAD,), f32),
        pltpu.SemaphoreType.DMA,
    ],
)
def _deg_pass(dst_hbm, ones_hbm, out_hbm, idv, onev, acc, sem):
    cid = lax.axis_index("c")
    sid = lax.axis_index("s")
    _zero_shared(onev.at[0], [acc])
    pltpu.sync_copy(ones_hbm, onev)
    pltpu.sync_copy(dst_hbm.at[pl.ds(_wid() * RPW, RPW)], idv)
    plsc.subcore_barrier()
    _drain(_scatter_add_rows(onev, idv, acc, sem, RPW))
    plsc.subcore_barrier()

    @pl.when(sid == 0)
    def _():
        pltpu.sync_copy(acc, out_hbm.at[pl.ds(cid * NPAD, NPAD)])


@functools.partial(
    pl.kernel,
    out_type=(_NODE_OUT, _NODE_OUT, _NODE_OUT, _NODE_OUT),
    mesh=_MESH,
    compiler_params=_CP,
    scratch_types=[
        pltpu.VMEM((NPAD,), f32),
        pltpu.VMEM((6 * 16,), f32),
        pltpu.VMEM((R1, 128), i32),
        pltpu.VMEM((R1, 128), i32),
        pltpu.VMEM((R1, 128), i32),
        pltpu.VMEM((R1, 128), f32),
        pltpu.VMEM((R1, 128), f32),
        pltpu.VMEM((R1, 128), f32),
        pltpu.VMEM((R1, 128), f32),
        pltpu.VMEM((R1, 128), f32),
        pltpu.VMEM((R1, 128), f32),
        pltpu.VMEM((R1, 128), f32),
        pltpu.VMEM((R1, 128), f32),
        pltpu.VMEM_SHARED((NPAD,), f32),
        pltpu.VMEM_SHARED((NPAD,), f32),
        pltpu.VMEM_SHARED((NPAD,), f32),
        pltpu.VMEM_SHARED((NPAD,), f32),
        pltpu.SemaphoreType.DMA,
    ],
)
def _layer1_pass(src_hbm, dst_hbm, deg_hbm, par_hbm,
                 s1p0_hbm, s1p1_hbm, ts0_hbm, ts1_hbm,
                 degv, parv, isv, idv_a, idv_b,
                 ex0_a, ex1_a, tv0_a, tv1_a, ex0_b, ex1_b, tv0_b, tv1_b,
                 acc_s0, acc_s1, acc_t0, acc_t1, sem):
    cid = lax.axis_index("c")
    sid = lax.axis_index("s")
    _zero_shared(degv, [acc_s0, acc_s1, acc_t0, acc_t1])
    pltpu.sync_copy(deg_hbm, degv)
    pltpu.sync_copy(par_hbm, parv)
    plsc.subcore_barrier()
    rowbase = _wid() * RPW
    cl0 = parv[pl.ds(0, 16)]
    cl1 = parv[pl.ds(16, 16)]
    cr0 = parv[pl.ds(32, 16)]
    cr1 = parv[pl.ds(48, 16)]
    m0 = parv[pl.ds(64, 16)]
    m1 = parv[pl.ds(80, 16)]
    sets = [(idv_a, ex0_a, ex1_a, tv0_a, tv1_a),
            (idv_b, ex0_b, ex1_b, tv0_b, tv1_b)]

    def load_compute(k, st):
        idv, ex0, ex1, tv0, tv1 = st
        rb = rowbase + k * R1
        pltpu.sync_copy(src_hbm.at[pl.ds(rb, R1)], isv)
        pltpu.sync_copy(dst_hbm.at[pl.ds(rb, R1)], idv)

        def row(j, cj):
            def lane(m, cm):
                sl = pl.ds(m * 16, 16)
                s16 = isv[j, sl]
                d16 = idv[j, sl]
                degs = plsc.load_gather(degv, [s16])
                degd = plsc.load_gather(degv, [d16])
                x0 = degs * cl0 + degd * cr0
                e0 = jnp.maximum(x0, x0 * _vc(0.2))
                v0 = jnp.exp(e0 - m0)
                x1 = degs * cl1 + degd * cr1
                e1 = jnp.maximum(x1, x1 * _vc(0.2))
                v1 = jnp.exp(e1 - m1)
                ex0[j, sl] = v0
                ex1[j, sl] = v1
                tv0[j, sl] = degs * v0
                tv1[j, sl] = degs * v1
                return cm

            lax.fori_loop(0, 8, lane, 0)
            return cj

        lax.fori_loop(0, R1, row, 0)

    load_compute(0, sets[0])
    for k in range(K1):
        idv, ex0, ex1, tv0, tv1 = sets[k % 2]
        ds_ = _scatter_add_rows(ex0, idv, acc_s0, sem, R1)
        ds_ += _scatter_add_rows(ex1, idv, acc_s1, sem, R1)
        ds_ += _scatter_add_rows(tv0, idv, acc_t0, sem, R1)
        ds_ += _scatter_add_rows(tv1, idv, acc_t1, sem, R1)
        if k + 1 < K1:
            load_compute(k + 1, sets[(k + 1) % 2])
        _drain(ds_)
    plsc.subcore_barrier()

    @pl.when(sid == 0)
    def _():
        sl = pl.ds(cid * NPAD, NPAD)
        pltpu.sync_copy(acc_s0, s1p0_hbm.at[sl])
        pltpu.sync_copy(acc_s1, s1p1_hbm.at[sl])
        pltpu.sync_copy(acc_t0, ts0_hbm.at[sl])
        pltpu.sync_copy(acc_t1, ts1_hbm.at[sl])


R2M = 16
K2M = (ER // NSUB) // R2M


@functools.partial(
    pl.kernel,
    out_type=(_NODE_OUT1, _NODE_OUT1, _NODE_OUT1,
              _NODE_OUT1, _NODE_OUT1, _NODE_OUT1),
    mesh=_MESH,
    compiler_params=_CP,
    scratch_types=[
        pltpu.VMEM((NPAD,), f32),
        pltpu.VMEM((NPAD,), f32),
        pltpu.VMEM((2 * 6 * 16,), f32),
        pltpu.VMEM((R2M, 128), i32),
        pltpu.VMEM((R2M, 128), i32),
        pltpu.VMEM((R2M, 128), f32),
        pltpu.VMEM((R2M, 128), f32),
        pltpu.VMEM((R2M, 128), f32),
        pltpu.VMEM_SHARED((NPAD,), f32),
        pltpu.VMEM_SHARED((NPAD,), f32),
        pltpu.VMEM_SHARED((NPAD,), f32),
        pltpu.SemaphoreType.DMA,
    ],
)
def _layer2_pass(src_hbm, dst_hbm, w0_hbm, w1_hbm, par_hbm,
                 s20_hbm, g00_hbm, g01_hbm, s21_hbm, g10_hbm, g11_hbm,
                 w0v, w1v, parv, isv, idv, exb, g0b, g1b,
                 acc_e, acc_g0, acc_g1, sem):
    cid = lax.axis_index("c")
    sid = lax.axis_index("s")
    _zero_shared(w0v, [acc_e, acc_g0, acc_g1])
    pltpu.sync_copy(w0_hbm, w0v)
    pltpu.sync_copy(w1_hbm, w1v)
    pltpu.sync_copy(par_hbm, parv)
    plsc.subcore_barrier()
    rowbase = sid * (ER // NSUB)
    pbase = cid * 96
    ca = parv[pl.ds(pbase, 16)]
    cb = parv[pl.ds(pbase + 16, 16)]
    cc = parv[pl.ds(pbase + 32, 16)]
    cd = parv[pl.ds(pbase + 48, 16)]
    cst = parv[pl.ds(pbase + 64, 16)]
    mh = parv[pl.ds(pbase + 80, 16)]

    def chunk(k, ck):
        rb = rowbase + k * R2M
        pltpu.sync_copy(src_hbm.at[pl.ds(rb, R2M)], isv)
        pltpu.sync_copy(dst_hbm.at[pl.ds(rb, R2M)], idv)

        def row(j, cj):
            def lane(m, cm):
                sl = pl.ds(m * 16, 16)
                s16 = isv[j, sl]
                d16 = idv[j, sl]
                w0s = plsc.load_gather(w0v, [s16])
                w1s = plsc.load_gather(w1v, [s16])
                w0d = plsc.load_gather(w0v, [d16])
                w1d = plsc.load_gather(w1v, [d16])
                x = w0s * ca + w1s * cb + w0d * cc + w1d * cd + cst
                e = jnp.maximum(x, x * _vc(0.2))
                v = jnp.exp(e - mh)
                exb[j, sl] = v
                g0b[j, sl] = v * w0s
                g1b[j, sl] = v * w1s
                return cm

            lax.fori_loop(0, 8, lane, 0)
            return cj

        lax.fori_loop(0, R2M, row, 0)
        ds_ = _scatter_add_rows(exb, idv, acc_e, sem, R2M)
        ds_ += _scatter_add_rows(g0b, idv, acc_g0, sem, R2M)
        ds_ += _scatter_add_rows(g1b, idv, acc_g1, sem, R2M)
        _drain(ds_)
        return ck

    lax.fori_loop(0, K2M, chunk, 0)
    plsc.subcore_barrier()

    @pl.when((sid == 0) & (cid == 0))
    def _():
        pltpu.sync_copy(acc_e, s20_hbm)
        pltpu.sync_copy(acc_g0, g00_hbm)
        pltpu.sync_copy(acc_g1, g01_hbm)

    @pl.when((sid == 0) & (cid == 1))
    def _():
        pltpu.sync_copy(acc_e, s21_hbm)
        pltpu.sync_copy(acc_g0, g10_hbm)
        pltpu.sync_copy(acc_g1, g11_hbm)


def _splat(vals):
    v = jnp.stack([v.astype(f32) for v in vals])
    return jnp.broadcast_to(v[:, None], (v.shape[0], 16)).reshape(-1)


def kernel(edge_index, W1, al1, ar1, b1, W2, al2, ar2, b2, Wc, bc):
    src = edge_index[0].astype(i32)
    dst = edge_index[1].astype(i32)
    pad = jnp.full((EPAD - E,), PADN, i32)
    src2 = jnp.concatenate([src, pad]).reshape(ER, 128)
    dst2 = jnp.concatenate([dst, pad]).reshape(ER, 128)
    ones = jnp.ones((RPW, 128), f32)

    def _comb(flat):
        return flat.reshape(NCORE, NPAD).sum(0)

    deg_valid = _comb(_deg_pass(dst2, ones))[:N]
    deg_pad = jnp.concatenate([deg_valid, jnp.zeros((NPAD - N,), f32)])

    w1r = W1.reshape(H, D)
    cl1 = (w1r * al1).sum(-1)
    cr1 = (w1r * ar1).sum(-1)
    maxdeg = deg_valid.max()
    m1 = maxdeg * (jax.nn.relu(cl1) + jax.nn.relu(cr1))
    par1 = _splat([cl1[0], cl1[1], cr1[0], cr1[1], m1[0], m1[1]])

    s1p0, s1p1, tsp0, tsp1 = _layer1_pass(src2, dst2, deg_pad, par1)
    w1s0 = _comb(tsp0) / jnp.maximum(_comb(s1p0), 1e-9)
    w1s1 = _comb(tsp1) / jnp.maximum(_comb(s1p1), 1e-9)
    w1 = jnp.stack([w1s0, w1s1], axis=1)

    A = w1r / H
    cvec = b1.mean(0)
    W2r = W2.reshape(D, H, D)
    vl2 = jnp.einsum("khd,hd->kh", W2r, al2)
    vr2 = jnp.einsum("khd,hd->kh", W2r, ar2)
    P = A @ vl2
    p0 = cvec @ vl2
    Q = A @ vr2
    q0 = cvec @ vr2
    el2 = w1 @ P + p0
    er2 = w1 @ Q + q0
    m2 = el2.max(0) + er2.max(0)
    m2 = jnp.maximum(m2, 0.2 * m2)

    par2 = jnp.concatenate([
        _splat([P[0, h], P[1, h], Q[0, h], Q[1, h], p0[h] + q0[h], m2[h]])
        for h in range(H)])
    s20, g00, g01, s21, g10, g11 = _layer2_pass(
        src2, dst2, w1s0, w1s1, par2)
    den0 = jnp.maximum(s20[:N], 1e-9)
    den1 = jnp.maximum(s21[:N], 1e-9)
    t = jnp.stack([(s20[:N] / den0).sum(), (s21[:N] / den1).sum()])
    S = jnp.stack([
        jnp.stack([(g00[:N] / den0).sum(), (g01[:N] / den0).sum()]),
        jnp.stack([(g10[:N] / den1).sum(), (g11[:N] / den1).sum()]),
    ])

    u = S @ A + t[:, None] * cvec[None, :]
    contrib = jnp.einsum("hk,khd->d", u, W2r)
    hg = contrib / (N * H) + b2.mean(0)
    return hg[None, :] @ Wc + bc


# speedup vs baseline: 474.4196x; 1.2656x over previous
import functools

import jax
import jax.numpy as jnp
from jax import lax
from jax.experimental import pallas as pl
from jax.experimental.pallas import tpu as pltpu
from jax.experimental.pallas import tpu_sc as plsc

N = 50000
E = 800000
H = 2
D = 32

NCORE = 2
NSUB = 16
NW = NCORE * NSUB

PER_W = 25600
EPAD = PER_W * NW
CH1 = 5120
K1 = PER_W // CH1
PER_C = EPAD // NSUB
CH2 = 2048
K2 = PER_C // CH2
NPAD = 50176
ZSL = NPAD // NSUB
PADN = N

f32 = jnp.float32
i32 = jnp.int32

_MESH = plsc.VectorSubcoreMesh(core_axis_name="c", subcore_axis_name="s")
_CP = pltpu.CompilerParams(needs_layout_passes=False)
_NODE_OUT = jax.ShapeDtypeStruct((NCORE * NPAD,), f32)
_NODE_OUT1 = jax.ShapeDtypeStruct((NPAD,), f32)


def _vc(v):
    return jnp.full((16,), v, f32)


def _wid():
    return lax.axis_index("c") * NSUB + lax.axis_index("s")


def _zero_shared(zsrc, accs):
    def zb(i, c):
        zsrc[pl.ds(i * 16, 16)] = jnp.zeros((16,), f32)
        return c

    lax.fori_loop(0, ZSL // 16, zb, 0)
    s = lax.axis_index("s")
    for acc in accs:
        pltpu.sync_copy(zsrc.at[pl.ds(0, ZSL)], acc.at[pl.ds(s * ZSL, ZSL)])


def _scatter_add(vals, idx, acc, sem):
    return [pltpu.async_copy(vals, acc.at[idx], sem, add=True)]


def _drain(descs):
    for d in descs:
        d.wait()


@functools.partial(
    pl.kernel,
    out_type=_NODE_OUT,
    mesh=_MESH,
    compiler_params=_CP,
    scratch_types=[
        pltpu.VMEM((PER_W,), i32),
        pltpu.VMEM((PER_W,), f32),
        pltpu.VMEM_SHARED((NPAD,), f32),
        pltpu.SemaphoreType.DMA,
    ],
)
def _deg_pass(dst_hbm, ones_hbm, out_hbm, idv, onev, acc, sem):
    cid = lax.axis_index("c")
    sid = lax.axis_index("s")
    _zero_shared(onev, [acc])
    pltpu.sync_copy(ones_hbm, onev)
    pltpu.sync_copy(dst_hbm.at[pl.ds(_wid() * PER_W, PER_W)], idv)
    plsc.subcore_barrier()
    _drain(_scatter_add(onev, idv, acc, sem))
    plsc.subcore_barrier()

    @pl.when(sid == 0)
    def _():
        pltpu.sync_copy(acc, out_hbm.at[pl.ds(cid * NPAD, NPAD)])


@functools.partial(
    pl.kernel,
    out_type=(_NODE_OUT, _NODE_OUT, _NODE_OUT, _NODE_OUT),
    mesh=_MESH,
    compiler_params=_CP,
    scratch_types=[
        pltpu.VMEM((NPAD,), f32),
        pltpu.VMEM((6 * 16,), f32),
        pltpu.VMEM((CH1,), i32),
        pltpu.VMEM((CH1,), i32),
        pltpu.VMEM((CH1,), i32),
        pltpu.VMEM((CH1,), f32),
        pltpu.VMEM((CH1,), f32),
        pltpu.VMEM((CH1,), f32),
        pltpu.VMEM((CH1,), f32),
        pltpu.VMEM((CH1,), f32),
        pltpu.VMEM((CH1,), f32),
        pltpu.VMEM((CH1,), f32),
        pltpu.VMEM((CH1,), f32),
        pltpu.VMEM_SHARED((NPAD,), f32),
        pltpu.VMEM_SHARED((NPAD,), f32),
        pltpu.VMEM_SHARED((NPAD,), f32),
        pltpu.VMEM_SHARED((NPAD,), f32),
        pltpu.SemaphoreType.DMA,
    ],
)
def _layer1_pass(src_hbm, dst_hbm, deg_hbm, par_hbm,
                 s1p0_hbm, s1p1_hbm, ts0_hbm, ts1_hbm,
                 degv, parv, isv, idv_a, idv_b,
                 ex0_a, ex1_a, tv0_a, tv1_a, ex0_b, ex1_b, tv0_b, tv1_b,
                 acc_s0, acc_s1, acc_t0, acc_t1, sem):
    cid = lax.axis_index("c")
    sid = lax.axis_index("s")
    _zero_shared(degv, [acc_s0, acc_s1, acc_t0, acc_t1])
    pltpu.sync_copy(deg_hbm, degv)
    pltpu.sync_copy(par_hbm, parv)
    plsc.subcore_barrier()
    ebase = _wid() * PER_W
    cl0 = parv[pl.ds(0, 16)]
    cl1 = parv[pl.ds(16, 16)]
    cr0 = parv[pl.ds(32, 16)]
    cr1 = parv[pl.ds(48, 16)]
    m0 = parv[pl.ds(64, 16)]
    m1 = parv[pl.ds(80, 16)]
    sets = [(idv_a, ex0_a, ex1_a, tv0_a, tv1_a),
            (idv_b, ex0_b, ex1_b, tv0_b, tv1_b)]

    def load_compute(k, st):
        idv, ex0, ex1, tv0, tv1 = st
        eb = ebase + k * CH1
        pltpu.sync_copy(src_hbm.at[pl.ds(eb, CH1)], isv)
        pltpu.sync_copy(dst_hbm.at[pl.ds(eb, CH1)], idv)

        def lane(m, cm):
            sl = pl.ds(m * 16, 16)
            s16 = isv[sl]
            d16 = idv[sl]
            degs = plsc.load_gather(degv, [s16])
            degd = plsc.load_gather(degv, [d16])
            x0 = degs * cl0 + degd * cr0
            e0 = jnp.maximum(x0, x0 * _vc(0.2))
            v0 = jnp.exp(e0 - m0)
            x1 = degs * cl1 + degd * cr1
            e1 = jnp.maximum(x1, x1 * _vc(0.2))
            v1 = jnp.exp(e1 - m1)
            ex0[sl] = v0
            ex1[sl] = v1
            tv0[sl] = degs * v0
            tv1[sl] = degs * v1
            return cm

        lax.fori_loop(0, CH1 // 16, lane, 0)

    load_compute(0, sets[0])
    for k in range(K1):
        idv, ex0, ex1, tv0, tv1 = sets[k % 2]
        ds_ = _scatter_add(ex0, idv, acc_s0, sem)
        ds_ += _scatter_add(ex1, idv, acc_s1, sem)
        ds_ += _scatter_add(tv0, idv, acc_t0, sem)
        ds_ += _scatter_add(tv1, idv, acc_t1, sem)
        if k + 1 < K1:
            load_compute(k + 1, sets[(k + 1) % 2])
        _drain(ds_)
    plsc.subcore_barrier()

    @pl.when(sid == 0)
    def _():
        sl = pl.ds(cid * NPAD, NPAD)
        pltpu.sync_copy(acc_s0, s1p0_hbm.at[sl])
        pltpu.sync_copy(acc_s1, s1p1_hbm.at[sl])
        pltpu.sync_copy(acc_t0, ts0_hbm.at[sl])
        pltpu.sync_copy(acc_t1, ts1_hbm.at[sl])


@functools.partial(
    pl.kernel,
    out_type=(_NODE_OUT1, _NODE_OUT1, _NODE_OUT1,
              _NODE_OUT1, _NODE_OUT1, _NODE_OUT1),
    mesh=_MESH,
    compiler_params=_CP,
    scratch_types=[
        pltpu.VMEM((NPAD,), f32),
        pltpu.VMEM((NPAD,), f32),
        pltpu.VMEM((2 * 6 * 16,), f32),
        pltpu.VMEM((CH2,), i32),
        pltpu.VMEM((CH2,), i32),
        pltpu.VMEM((CH2,), i32),
        pltpu.VMEM((CH2,), f32),
        pltpu.VMEM((CH2,), f32),
        pltpu.VMEM((CH2,), f32),
        pltpu.VMEM((CH2,), f32),
        pltpu.VMEM((CH2,), f32),
        pltpu.VMEM((CH2,), f32),
        pltpu.VMEM_SHARED((NPAD,), f32),
        pltpu.VMEM_SHARED((NPAD,), f32),
        pltpu.VMEM_SHARED((NPAD,), f32),
        pltpu.SemaphoreType.DMA,
    ],
)
def _layer2_pass(src_hbm, dst_hbm, w0_hbm, w1_hbm, par_hbm,
                 s20_hbm, g00_hbm, g01_hbm, s21_hbm, g10_hbm, g11_hbm,
                 w0v, w1v, parv, isv, idv_a, idv_b,
                 ex_a, g0_a, g1_a, ex_b, g0_b, g1_b,
                 acc_e, acc_g0, acc_g1, sem):
    cid = lax.axis_index("c")
    sid = lax.axis_index("s")
    _zero_shared(w0v, [acc_e, acc_g0, acc_g1])
    pltpu.sync_copy(w0_hbm, w0v)
    pltpu.sync_copy(w1_hbm, w1v)
    pltpu.sync_copy(par_hbm, parv)
    plsc.subcore_barrier()
    ebase = sid * PER_C
    pbase = cid * 96
    ca = parv[pl.ds(pbase, 16)]
    cb = parv[pl.ds(pbase + 16, 16)]
    cc = parv[pl.ds(pbase + 32, 16)]
    cd = parv[pl.ds(pbase + 48, 16)]
    cst = parv[pl.ds(pbase + 64, 16)]
    mh = parv[pl.ds(pbase + 80, 16)]
    sets = [(idv_a, ex_a, g0_a, g1_a), (idv_b, ex_b, g0_b, g1_b)]

    def load_compute(k, st):
        idv, exb, g0b, g1b = st
        eb = ebase + k * CH2
        pltpu.sync_copy(src_hbm.at[pl.ds(eb, CH2)], isv)
        pltpu.sync_copy(dst_hbm.at[pl.ds(eb, CH2)], idv)

        def lane(m, cm):
            sl = pl.ds(m * 16, 16)
            s16 = isv[sl]
            d16 = idv[sl]
            w0s = plsc.load_gather(w0v, [s16])
            w1s = plsc.load_gather(w1v, [s16])
            w0d = plsc.load_gather(w0v, [d16])
            w1d = plsc.load_gather(w1v, [d16])
            x = w0s * ca + w1s * cb + w0d * cc + w1d * cd + cst
            e = jnp.maximum(x, x * _vc(0.2))
            v = jnp.exp(e - mh)
            exb[sl] = v
            g0b[sl] = v * w0s
            g1b[sl] = v * w1s
            return cm

        lax.fori_loop(0, CH2 // 16, lane, 0)

    load_compute(0, sets[0])
    for k in range(K2):
        idv, exb, g0b, g1b = sets[k % 2]
        ds_ = _scatter_add(exb, idv, acc_e, sem)
        ds_ += _scatter_add(g0b, idv, acc_g0, sem)
        ds_ += _scatter_add(g1b, idv, acc_g1, sem)
        if k + 1 < K2:
            load_compute(k + 1, sets[(k + 1) % 2])
        _drain(ds_)
    plsc.subcore_barrier()

    @pl.when((sid == 0) & (cid == 0))
    def _():
        pltpu.sync_copy(acc_e, s20_hbm)
        pltpu.sync_copy(acc_g0, g00_hbm)
        pltpu.sync_copy(acc_g1, g01_hbm)

    @pl.when((sid == 0) & (cid == 1))
    def _():
        pltpu.sync_copy(acc_e, s21_hbm)
        pltpu.sync_copy(acc_g0, g10_hbm)
        pltpu.sync_copy(acc_g1, g11_hbm)


def _splat(vals):
    v = jnp.stack([v.astype(f32) for v in vals])
    return jnp.broadcast_to(v[:, None], (v.shape[0], 16)).reshape(-1)


def kernel(edge_index, W1, al1, ar1, b1, W2, al2, ar2, b2, Wc, bc):
    src = edge_index[0].astype(i32)
    dst = edge_index[1].astype(i32)
    pad = jnp.full((EPAD - E,), PADN, i32)
    src1 = jnp.concatenate([src, pad])
    dst1 = jnp.concatenate([dst, pad])
    ones = jnp.ones((PER_W,), f32)

    def _comb(flat):
        return flat.reshape(NCORE, NPAD).sum(0)

    deg_valid = _comb(_deg_pass(dst1, ones))[:N]
    deg_pad = jnp.concatenate([deg_valid, jnp.zeros((NPAD - N,), f32)])

    w1r = W1.reshape(H, D)
    cl1 = (w1r * al1).sum(-1)
    cr1 = (w1r * ar1).sum(-1)
    maxdeg = deg_valid.max()
    m1 = maxdeg * (jax.nn.relu(cl1) + jax.nn.relu(cr1))
    par1 = _splat([cl1[0], cl1[1], cr1[0], cr1[1], m1[0], m1[1]])

    s1p0, s1p1, tsp0, tsp1 = _layer1_pass(src1, dst1, deg_pad, par1)
    w1s0 = _comb(tsp0) / jnp.maximum(_comb(s1p0), 1e-9)
    w1s1 = _comb(tsp1) / jnp.maximum(_comb(s1p1), 1e-9)
    w1 = jnp.stack([w1s0, w1s1], axis=1)

    A = w1r / H
    cvec = b1.mean(0)
    W2r = W2.reshape(D, H, D)
    vl2 = jnp.einsum("khd,hd->kh", W2r, al2)
    vr2 = jnp.einsum("khd,hd->kh", W2r, ar2)
    P = A @ vl2
    p0 = cvec @ vl2
    Q = A @ vr2
    q0 = cvec @ vr2
    el2 = w1 @ P + p0
    er2 = w1 @ Q + q0
    m2 = el2.max(0) + er2.max(0)
    m2 = jnp.maximum(m2, 0.2 * m2)

    par2 = jnp.concatenate([
        _splat([P[0, h], P[1, h], Q[0, h], Q[1, h], p0[h] + q0[h], m2[h]])
        for h in range(H)])
    s20, g00, g01, s21, g10, g11 = _layer2_pass(src1, dst1, w1s0, w1s1, par2)
    den0 = jnp.maximum(s20[:N], 1e-9)
    den1 = jnp.maximum(s21[:N], 1e-9)
    t = jnp.stack([(s20[:N] / den0).sum(), (s21[:N] / den1).sum()])
    S = jnp.stack([
        jnp.stack([(g00[:N] / den0).sum(), (g01[:N] / den0).sum()]),
        jnp.stack([(g10[:N] / den1).sum(), (g11[:N] / den1).sum()]),
    ])

    u = S @ A + t[:, None] * cvec[None, :]
    contrib = jnp.einsum("hk,khd->d", u, W2r)
    hg = contrib / (N * H) + b2.mean(0)
    return hg[None, :] @ Wc + bc


# speedup vs baseline: 479.4704x; 1.0106x over previous
import functools

import jax
import jax.numpy as jnp
from jax import lax
from jax.experimental import pallas as pl
from jax.experimental.pallas import tpu as pltpu
from jax.experimental.pallas import tpu_sc as plsc

N = 50000
E = 800000
H = 2
D = 32

NCORE = 2
NSUB = 16
NW = NCORE * NSUB

PER_W = 25600
EPAD = PER_W * NW
CH1 = 5120
K1 = PER_W // CH1
PER_C = EPAD // NSUB
CH2 = 2048
K2 = PER_C // CH2
NPAD = 50176
ZSL = NPAD // NSUB
PADN = N

f32 = jnp.float32
i32 = jnp.int32

_MESH = plsc.VectorSubcoreMesh(core_axis_name="c", subcore_axis_name="s")
_CP = pltpu.CompilerParams(needs_layout_passes=False)
_NODE_OUT1 = jax.ShapeDtypeStruct((NPAD,), f32)


def _vc(v):
    return jnp.full((16,), v, f32)


def _wid():
    return lax.axis_index("c") * NSUB + lax.axis_index("s")


def _zero_shared(zsrc, accs):
    def zb(i, c):
        zsrc[pl.ds(i * 16, 16)] = jnp.zeros((16,), f32)
        return c

    lax.fori_loop(0, ZSL // 16, zb, 0)
    s = lax.axis_index("s")
    for acc in accs:
        pltpu.sync_copy(zsrc.at[pl.ds(0, ZSL)], acc.at[pl.ds(s * ZSL, ZSL)])


def _scatter_add(vals, idx, acc, sem):
    return [pltpu.async_copy(vals, acc.at[idx], sem, add=True)]


def _drain(descs):
    for d in descs:
        d.wait()


@functools.partial(
    pl.kernel,
    out_type=(_NODE_OUT1, _NODE_OUT1),
    mesh=_MESH,
    compiler_params=_CP,
    scratch_types=[
        pltpu.VMEM((PER_W,), i32),
        pltpu.VMEM((PER_W,), f32),
        pltpu.VMEM_SHARED((NPAD,), f32),
        pltpu.SemaphoreType.DMA,
    ],
)
def _deg_pass(dst_hbm, ones_hbm, out0_hbm, out1_hbm, idv, onev, acc, sem):
    cid = lax.axis_index("c")
    sid = lax.axis_index("s")
    _zero_shared(onev, [acc])
    pltpu.sync_copy(ones_hbm, onev)
    pltpu.sync_copy(dst_hbm.at[pl.ds(_wid() * PER_W, PER_W)], idv)
    plsc.subcore_barrier()
    _drain(_scatter_add(onev, idv, acc, sem))
    plsc.subcore_barrier()

    @pl.when((sid == 0) & (cid == 0))
    def _():
        pltpu.sync_copy(acc, out0_hbm)

    @pl.when((sid == 0) & (cid == 1))
    def _():
        pltpu.sync_copy(acc, out1_hbm)


@functools.partial(
    pl.kernel,
    out_type=(_NODE_OUT1,) * 8,
    mesh=_MESH,
    compiler_params=_CP,
    scratch_types=[
        pltpu.VMEM((NPAD,), f32),
        pltpu.VMEM((6 * 16,), f32),
        pltpu.VMEM((CH1,), i32),
        pltpu.VMEM((CH1,), i32),
        pltpu.VMEM((CH1,), i32),
        pltpu.VMEM((CH1,), f32),
        pltpu.VMEM((CH1,), f32),
        pltpu.VMEM((CH1,), f32),
        pltpu.VMEM((CH1,), f32),
        pltpu.VMEM((CH1,), f32),
        pltpu.VMEM((CH1,), f32),
        pltpu.VMEM((CH1,), f32),
        pltpu.VMEM((CH1,), f32),
        pltpu.VMEM_SHARED((NPAD,), f32),
        pltpu.VMEM_SHARED((NPAD,), f32),
        pltpu.VMEM_SHARED((NPAD,), f32),
        pltpu.VMEM_SHARED((NPAD,), f32),
        pltpu.SemaphoreType.DMA,
    ],
)
def _layer1_pass(src_hbm, dst_hbm, deg_hbm, par_hbm,
                 s1a_hbm, s1b_hbm, tsa_hbm, tsb_hbm,
                 s1a2_hbm, s1b2_hbm, tsa2_hbm, tsb2_hbm,
                 degv, parv, isv, idv_a, idv_b,
                 ex0_a, ex1_a, tv0_a, tv1_a, ex0_b, ex1_b, tv0_b, tv1_b,
                 acc_s0, acc_s1, acc_t0, acc_t1, sem):
    cid = lax.axis_index("c")
    sid = lax.axis_index("s")
    _zero_shared(degv, [acc_s0, acc_s1, acc_t0, acc_t1])
    pltpu.sync_copy(deg_hbm, degv)
    pltpu.sync_copy(par_hbm, parv)
    plsc.subcore_barrier()
    ebase = _wid() * PER_W
    cl0 = parv[pl.ds(0, 16)]
    cl1 = parv[pl.ds(16, 16)]
    cr0 = parv[pl.ds(32, 16)]
    cr1 = parv[pl.ds(48, 16)]
    m0 = parv[pl.ds(64, 16)]
    m1 = parv[pl.ds(80, 16)]
    sets = [(idv_a, ex0_a, ex1_a, tv0_a, tv1_a),
            (idv_b, ex0_b, ex1_b, tv0_b, tv1_b)]

    def load_compute(k, st):
        idv, ex0, ex1, tv0, tv1 = st
        eb = ebase + k * CH1
        pltpu.sync_copy(src_hbm.at[pl.ds(eb, CH1)], isv)
        pltpu.sync_copy(dst_hbm.at[pl.ds(eb, CH1)], idv)

        def lane(m, cm):
            sl = pl.ds(m * 16, 16)
            s16 = isv[sl]
            d16 = idv[sl]
            degs = plsc.load_gather(degv, [s16])
            degd = plsc.load_gather(degv, [d16])
            x0 = degs * cl0 + degd * cr0
            e0 = jnp.maximum(x0, x0 * _vc(0.2))
            v0 = jnp.exp(e0 - m0)
            x1 = degs * cl1 + degd * cr1
            e1 = jnp.maximum(x1, x1 * _vc(0.2))
            v1 = jnp.exp(e1 - m1)
            ex0[sl] = v0
            ex1[sl] = v1
            tv0[sl] = degs * v0
            tv1[sl] = degs * v1
            return cm

        lax.fori_loop(0, CH1 // 16, lane, 0)

    load_compute(0, sets[0])
    for k in range(K1):
        idv, ex0, ex1, tv0, tv1 = sets[k % 2]
        ds_ = _scatter_add(ex0, idv, acc_s0, sem)
        ds_ += _scatter_add(ex1, idv, acc_s1, sem)
        ds_ += _scatter_add(tv0, idv, acc_t0, sem)
        ds_ += _scatter_add(tv1, idv, acc_t1, sem)
        if k + 1 < K1:
            load_compute(k + 1, sets[(k + 1) % 2])
        _drain(ds_)
    plsc.subcore_barrier()

    @pl.when((sid == 0) & (cid == 0))
    def _():
        pltpu.sync_copy(acc_s0, s1a_hbm)
        pltpu.sync_copy(acc_s1, s1b_hbm)
        pltpu.sync_copy(acc_t0, tsa_hbm)
        pltpu.sync_copy(acc_t1, tsb_hbm)

    @pl.when((sid == 0) & (cid == 1))
    def _():
        pltpu.sync_copy(acc_s0, s1a2_hbm)
        pltpu.sync_copy(acc_s1, s1b2_hbm)
        pltpu.sync_copy(acc_t0, tsa2_hbm)
        pltpu.sync_copy(acc_t1, tsb2_hbm)


@functools.partial(
    pl.kernel,
    out_type=(_NODE_OUT1, _NODE_OUT1, _NODE_OUT1,
              _NODE_OUT1, _NODE_OUT1, _NODE_OUT1),
    mesh=_MESH,
    compiler_params=_CP,
    scratch_types=[
        pltpu.VMEM((NPAD,), f32),
        pltpu.VMEM((NPAD,), f32),
        pltpu.VMEM((2 * 6 * 16,), f32),
        pltpu.VMEM((CH2,), i32),
        pltpu.VMEM((CH2,), i32),
        pltpu.VMEM((CH2,), i32),
        pltpu.VMEM((CH2,), f32),
        pltpu.VMEM((CH2,), f32),
        pltpu.VMEM((CH2,), f32),
        pltpu.VMEM((CH2,), f32),
        pltpu.VMEM((CH2,), f32),
        pltpu.VMEM((CH2,), f32),
        pltpu.VMEM_SHARED((NPAD,), f32),
        pltpu.VMEM_SHARED((NPAD,), f32),
        pltpu.VMEM_SHARED((NPAD,), f32),
        pltpu.SemaphoreType.DMA,
    ],
)
def _layer2_pass(src_hbm, dst_hbm, w0_hbm, w1_hbm, par_hbm,
                 s20_hbm, g00_hbm, g01_hbm, s21_hbm, g10_hbm, g11_hbm,
                 w0v, w1v, parv, isv, idv_a, idv_b,
                 ex_a, g0_a, g1_a, ex_b, g0_b, g1_b,
                 acc_e, acc_g0, acc_g1, sem):
    cid = lax.axis_index("c")
    sid = lax.axis_index("s")
    _zero_shared(w0v, [acc_e, acc_g0, acc_g1])
    pltpu.sync_copy(w0_hbm, w0v)
    pltpu.sync_copy(w1_hbm, w1v)
    pltpu.sync_copy(par_hbm, parv)
    plsc.subcore_barrier()
    ebase = sid * PER_C
    pbase = cid * 96
    ca = parv[pl.ds(pbase, 16)]
    cb = parv[pl.ds(pbase + 16, 16)]
    cc = parv[pl.ds(pbase + 32, 16)]
    cd = parv[pl.ds(pbase + 48, 16)]
    cst = parv[pl.ds(pbase + 64, 16)]
    mh = parv[pl.ds(pbase + 80, 16)]
    sets = [(idv_a, ex_a, g0_a, g1_a), (idv_b, ex_b, g0_b, g1_b)]

    def load_compute(k, st):
        idv, exb, g0b, g1b = st
        eb = ebase + k * CH2
        pltpu.sync_copy(src_hbm.at[pl.ds(eb, CH2)], isv)
        pltpu.sync_copy(dst_hbm.at[pl.ds(eb, CH2)], idv)

        def lane(m, cm):
            sl = pl.ds(m * 16, 16)
            s16 = isv[sl]
            d16 = idv[sl]
            w0s = plsc.load_gather(w0v, [s16])
            w1s = plsc.load_gather(w1v, [s16])
            w0d = plsc.load_gather(w0v, [d16])
            w1d = plsc.load_gather(w1v, [d16])
            x = w0s * ca + w1s * cb + w0d * cc + w1d * cd + cst
            e = jnp.maximum(x, x * _vc(0.2))
            v = jnp.exp(e - mh)
            exb[sl] = v
            g0b[sl] = v * w0s
            g1b[sl] = v * w1s
            return cm

        lax.fori_loop(0, CH2 // 16, lane, 0)

    load_compute(0, sets[0])
    for k in range(K2):
        idv, exb, g0b, g1b = sets[k % 2]
        ds_ = _scatter_add(exb, idv, acc_e, sem)
        ds_ += _scatter_add(g0b, idv, acc_g0, sem)
        ds_ += _scatter_add(g1b, idv, acc_g1, sem)
        if k + 1 < K2:
            load_compute(k + 1, sets[(k + 1) % 2])
        _drain(ds_)
    plsc.subcore_barrier()

    @pl.when((sid == 0) & (cid == 0))
    def _():
        pltpu.sync_copy(acc_e, s20_hbm)
        pltpu.sync_copy(acc_g0, g00_hbm)
        pltpu.sync_copy(acc_g1, g01_hbm)

    @pl.when((sid == 0) & (cid == 1))
    def _():
        pltpu.sync_copy(acc_e, s21_hbm)
        pltpu.sync_copy(acc_g0, g10_hbm)
        pltpu.sync_copy(acc_g1, g11_hbm)


def _splat(vals):
    v = jnp.stack([v.astype(f32) for v in vals])
    return jnp.broadcast_to(v[:, None], (v.shape[0], 16)).reshape(-1)


def kernel(edge_index, W1, al1, ar1, b1, W2, al2, ar2, b2, Wc, bc):
    src = edge_index[0].astype(i32)
    dst = edge_index[1].astype(i32)
    pad = jnp.full((EPAD - E,), PADN, i32)
    src1 = jnp.concatenate([src, pad])
    dst1 = jnp.concatenate([dst, pad])
    ones = jnp.ones((PER_W,), f32)

    dp0, dp1 = _deg_pass(dst1, ones)
    deg_valid = (dp0 + dp1)[:N]
    deg_pad = jnp.concatenate([deg_valid, jnp.zeros((NPAD - N,), f32)])

    w1r = W1.reshape(H, D)
    cl1 = (w1r * al1).sum(-1)
    cr1 = (w1r * ar1).sum(-1)
    maxdeg = deg_valid.max()
    m1 = maxdeg * (jax.nn.relu(cl1) + jax.nn.relu(cr1))
    par1 = _splat([cl1[0], cl1[1], cr1[0], cr1[1], m1[0], m1[1]])

    (s1a, s1b, tsa, tsb,
     s1a2, s1b2, tsa2, tsb2) = _layer1_pass(src1, dst1, deg_pad, par1)
    w1s0 = (tsa + tsa2) / jnp.maximum(s1a + s1a2, 1e-9)
    w1s1 = (tsb + tsb2) / jnp.maximum(s1b + s1b2, 1e-9)
    w1 = jnp.stack([w1s0, w1s1], axis=1)

    A = w1r / H
    cvec = b1.mean(0)
    W2r = W2.reshape(D, H, D)
    vl2 = jnp.einsum("khd,hd->kh", W2r, al2)
    vr2 = jnp.einsum("khd,hd->kh", W2r, ar2)
    P = A @ vl2
    p0 = cvec @ vl2
    Q = A @ vr2
    q0 = cvec @ vr2
    el2 = w1 @ P + p0
    er2 = w1 @ Q + q0
    m2 = el2.max(0) + er2.max(0)
    m2 = jnp.maximum(m2, 0.2 * m2)

    par2 = jnp.concatenate([
        _splat([P[0, h], P[1, h], Q[0, h], Q[1, h], p0[h] + q0[h], m2[h]])
        for h in range(H)])
    s20, g00, g01, s21, g10, g11 = _layer2_pass(src1, dst1, w1s0, w1s1, par2)
    den0 = jnp.maximum(s20[:N], 1e-9)
    den1 = jnp.maximum(s21[:N], 1e-9)
    t = jnp.stack([(s20[:N] / den0).sum(), (s21[:N] / den1).sum()])
    S = jnp.stack([
        jnp.stack([(g00[:N] / den0).sum(), (g01[:N] / den0).sum()]),
        jnp.stack([(g10[:N] / den1).sum(), (g11[:N] / den1).sum()]),
    ])

    u = S @ A + t[:, None] * cvec[None, :]
    contrib = jnp.einsum("hk,khd->d", u, W2r)
    hg = contrib / (N * H) + b2.mean(0)
    return hg[None, :] @ Wc + bc
